# baseline (speedup 1.0000x reference)
import jax
import jax.numpy as jnp
from jax.experimental import pallas as pl
from jax.experimental.pallas import tpu as pltpu

B = 2048
D = 128
E = 64
R = 256
K = 8
G = 4


def _gate_kernel(h_ref, u_ref, U_ref, V_ref, bias_ref, out_ref, g_ref):
    i = pl.program_id(0)
    h = h_ref[...]
    u = u_ref[...]

    for j in range(G):
        Uj = U_ref[j * R:(j + 1) * R, :]
        Vj = V_ref[j * R:(j + 1) * R, :]
        hUT = jax.lax.dot_general(Uj, h, (((1,), (1,)), ((), ())),
                                  preferred_element_type=jnp.float32)
        uVT = jax.lax.dot_general(Vj, u, (((1,), (1,)), ((), ())),
                                  preferred_element_type=jnp.float32)
        pj = hUT * uVT
        g_ref[pl.ds(i * G + j, 1), :] = jnp.sum(pj, axis=0, keepdims=True)

    @pl.when(i == (E // G) - 1)
    def _():
        x = g_ref[...] + bias_ref[...]
        cur = x
        for _ in range(K - 1):
            m = jnp.max(cur, axis=0, keepdims=True)
            cur = jnp.where(cur >= m, -jnp.inf, cur)
        t8 = jnp.max(cur, axis=0, keepdims=True)
        sel = x >= t8
        xm = jnp.max(x, axis=0, keepdims=True)
        ex = jnp.where(sel, jnp.exp(x - xm), 0.0)
        w = ex / jnp.sum(ex, axis=0, keepdims=True)
        out_ref[...] = jax.lax.transpose(w, (1, 0))


def kernel(h, u, U, V, bias):
    Ur = U.reshape(E * R, D)
    Vr = V.reshape(E * R, D)
    bias2 = bias.reshape(E, 1)
    return pl.pallas_call(
        _gate_kernel,
        grid=(E // G,),
        in_specs=[
            pl.BlockSpec((B, D), lambda i: (0, 0)),
            pl.BlockSpec((B, D), lambda i: (0, 0)),
            pl.BlockSpec((G * R, D), lambda i: (i, 0)),
            pl.BlockSpec((G * R, D), lambda i: (i, 0)),
            pl.BlockSpec((E, 1), lambda i: (0, 0)),
        ],
        out_specs=pl.BlockSpec((B, E), lambda i: (0, 0)),
        out_shape=jax.ShapeDtypeStruct((B, E), jnp.float32),
        scratch_shapes=[pltpu.VMEM((E, B), jnp.float32)],
    )(h, u, Ur, Vr, bias2)
